# baseline (speedup 1.0000x reference)
import functools

import jax
import jax.numpy as jnp
from jax import lax
from jax.experimental import pallas as pl
from jax.experimental.pallas import tpu as pltpu
from jax.experimental.pallas import tpu_sc as plsc

N_NODES = 10000
D_FEAT = 128
N_EDGES = 320000

NC = 2
NS = 16
NW = NC * NS

EDGE_B = 128
BATCH_PER_W = 80
N_BATCH = NW * BATCH_PER_W
E_PAD = N_BATCH * EDGE_B
CHUNK_B = 16
N_CHUNKS_W = BATCH_PER_W // CHUNK_B
PAIRS_PER_CHUNK = CHUNK_B // 2
ACC_ROWS = N_NODES + EDGE_B

ROW_CHUNK = 80
N_CHUNK = N_NODES // ROW_CHUNK
CHUNK_PER_S = -(-N_CHUNK // NS)


def _sc_partial(x, src4d, dst4d):
    mesh = plsc.VectorSubcoreMesh(core_axis_name="c", subcore_axis_name="s")

    scratch = dict(
        rows=pltpu.VMEM((EDGE_B, D_FEAT), jnp.float32),
        rows2=pltpu.VMEM((EDGE_B, D_FEAT), jnp.float32),
        acc=pltpu.VMEM_SHARED((ACC_ROWS, D_FEAT), jnp.float32),
        gsem=pltpu.SemaphoreType.DMA,
        gsem2=pltpu.SemaphoreType.DMA,
        ssem=pltpu.SemaphoreType.DMA,
        ssem2=pltpu.SemaphoreType.DMA,
    )
    for par in "AB":
        scratch[f"sbuf{par}"] = pltpu.VMEM((CHUNK_B, 1, EDGE_B), jnp.int32)
        scratch[f"dbuf{par}"] = pltpu.VMEM((CHUNK_B, 1, EDGE_B), jnp.int32)
        scratch[f"csem{par}"] = pltpu.SemaphoreType.DMA
        scratch[f"cdsem{par}"] = pltpu.SemaphoreType.DMA

    @functools.partial(
        pl.kernel,
        out_type=jax.ShapeDtypeStruct((NC, N_NODES, D_FEAT), jnp.float32),
        mesh=mesh,
        scratch_types=scratch,
    )
    def kern(x_hbm, s_hbm, d_hbm, part_hbm, *, rows, rows2, acc,
             gsem, gsem2, ssem, ssem2, **cbufs):
        c = lax.axis_index("c")
        s = lax.axis_index("s")
        w = c * NS + s
        ck0 = w * N_CHUNKS_W
        rowbuf = [rows, rows2]
        rsem = [gsem, gsem2]
        wsem = [ssem, ssem2]

        def chunk_fire(k, par):
            pltpu.async_copy(s_hbm.at[ck0 + k], cbufs[f"sbuf{par}"],
                             cbufs[f"csem{par}"])
            pltpu.async_copy(d_hbm.at[ck0 + k], cbufs[f"dbuf{par}"],
                             cbufs[f"cdsem{par}"])

        def chunk_wait(par):
            pltpu.make_async_copy(s_hbm.at[0], cbufs[f"sbuf{par}"],
                                  cbufs[f"csem{par}"]).wait()
            pltpu.make_async_copy(d_hbm.at[0], cbufs[f"dbuf{par}"],
                                  cbufs[f"cdsem{par}"]).wait()

        def scat_drain():
            for b in range(2):
                pltpu.make_async_copy(x_hbm.at[pl.ds(0, EDGE_B), :],
                                      rowbuf[b], wsem[b]).wait()

        chunk_fire(0, "A")

        zero = jnp.zeros((16,), jnp.float32)

        def zrow(r, _):
            def zcol(kk, _):
                rows[r, pl.ds(kk * 16, 16)] = zero
                return 0
            return lax.fori_loop(0, D_FEAT // 16, zcol, 0)

        lax.fori_loop(0, ROW_CHUNK, zrow, 0)

        def zchunk(i, _):
            ch = s + i * NS

            @pl.when(ch < N_CHUNK)
            def _():
                pltpu.async_copy(rows.at[pl.ds(0, ROW_CHUNK), :],
                                 acc.at[pl.ds(ch * ROW_CHUNK, ROW_CHUNK), :],
                                 ssem)
            return 0

        lax.fori_loop(0, CHUNK_PER_S, zchunk, 0)

        @pl.when(s == 0)
        def _():
            pltpu.async_copy(rows.at[pl.ds(0, ROW_CHUNK), :],
                             acc.at[pl.ds(N_NODES, ROW_CHUNK), :], ssem)

        def zdrain(i, _):
            ch = s + i * NS

            @pl.when(ch < N_CHUNK)
            def _():
                pltpu.make_async_copy(rows.at[pl.ds(0, ROW_CHUNK), :],
                                      acc.at[pl.ds(0, ROW_CHUNK), :],
                                      ssem).wait()
            return 0

        lax.fori_loop(0, CHUNK_PER_S, zdrain, 0)

        @pl.when(s == 0)
        def _():
            pltpu.make_async_copy(rows.at[pl.ds(0, ROW_CHUNK), :],
                                  acc.at[pl.ds(0, ROW_CHUNK), :], ssem).wait()
        plsc.subcore_barrier()

        def make_pair(par, first_chunk, next_par):
            sb = cbufs[f"sbuf{par}"]
            db = cbufs[f"dbuf{par}"]

            def pair(p, k):
                if first_chunk:
                    @pl.when(p > 0)
                    def _():
                        scat_drain()
                else:
                    scat_drain()

                @pl.when(p == 0)
                def _():
                    @pl.when(k + 1 < N_CHUNKS_W)
                    def _():
                        chunk_fire(k + 1, next_par)

                gets = [
                    pltpu.async_copy(x_hbm.at[sb.at[2 * p + b, 0]],
                                     rowbuf[b], rsem[b])
                    for b in range(2)
                ]
                for b in range(2):
                    gets[b].wait()
                    pltpu.async_copy(rowbuf[b], acc.at[db.at[2 * p + b, 0]],
                                     wsem[b], add=True)

            return pair

        for k in range(N_CHUNKS_W):
            par = "AB"[k % 2]
            nxt = "AB"[(k + 1) % 2]
            chunk_wait(par)
            pair_fn = make_pair(par, k == 0, nxt)

            def body(p, _, fn=pair_fn, kk=k):
                fn(p, kk)
                return 0

            lax.fori_loop(0, PAIRS_PER_CHUNK, body, 0)

        scat_drain()
        plsc.subcore_barrier()

        def wchunk(i, _):
            ch = s + i * NS

            @pl.when(ch < N_CHUNK)
            def _():
                r0 = ch * ROW_CHUNK
                pltpu.async_copy(
                    acc.at[pl.ds(r0, ROW_CHUNK), :],
                    part_hbm.at[c, pl.ds(r0, ROW_CHUNK), :],
                    ssem,
                )
            return 0

        lax.fori_loop(0, CHUNK_PER_S, wchunk, 0)

        def wdrain(i, _):
            ch = s + i * NS

            @pl.when(ch < N_CHUNK)
            def _():
                pltpu.make_async_copy(
                    acc.at[pl.ds(0, ROW_CHUNK), :],
                    part_hbm.at[c, pl.ds(0, ROW_CHUNK), :],
                    ssem,
                ).wait()
            return 0

        lax.fori_loop(0, CHUNK_PER_S, wdrain, 0)

    return kern(x, src4d, dst4d)


def _combine(parts):
    blk = 400

    def body(p_ref, o_ref):
        o_ref[...] = p_ref[0] + p_ref[1]

    return pl.pallas_call(
        body,
        grid=(N_NODES // blk,),
        in_specs=[pl.BlockSpec((NC, blk, D_FEAT), lambda i: (0, i, 0))],
        out_specs=pl.BlockSpec((blk, D_FEAT), lambda i: (i, 0)),
        out_shape=jax.ShapeDtypeStruct((N_NODES, D_FEAT), jnp.float32),
    )(parts)


def kernel(x, edge_index):
    ei = edge_index.astype(jnp.int32)
    n_pad = E_PAD - N_EDGES
    src = jnp.concatenate([ei[0], jnp.zeros((n_pad,), jnp.int32)])
    pad_dst = N_NODES + (jnp.arange(n_pad, dtype=jnp.int32) % EDGE_B)
    dst = jnp.concatenate([ei[1], pad_dst])
    n_ck = N_BATCH // CHUNK_B
    src4d = src.reshape(n_ck, CHUNK_B, 1, EDGE_B)
    dst4d = dst.reshape(n_ck, CHUNK_B, 1, EDGE_B)
    parts = _sc_partial(x, src4d, dst4d)
    return _combine(parts)


# speedup vs baseline: 3.0275x; 3.0275x over previous
import functools

import jax
import jax.numpy as jnp
from jax import lax
from jax.experimental import pallas as pl
from jax.experimental.pallas import tpu as pltpu
from jax.experimental.pallas import tpu_sc as plsc

N_NODES = 10000
D_FEAT = 128
N_EDGES = 320000

NC = 2
NS = 16
NW = NC * NS

EDGE_B = 128
BATCH_PER_W = 80
N_BATCH = NW * BATCH_PER_W
E_PAD = N_BATCH * EDGE_B
N_REAL_BATCH = N_EDGES // EDGE_B
CHUNK_B = 16
N_CHUNKS_W = BATCH_PER_W // CHUNK_B
PAIRS_PER_CHUNK = CHUNK_B // 2
ACC_ROWS = N_NODES + EDGE_B

ROW_CHUNK = 80
N_CHUNK = N_NODES // ROW_CHUNK
CHUNK_PER_S = -(-N_CHUNK // NS)


def _sc_partial(x, src4d, dst4d):
    mesh = plsc.VectorSubcoreMesh(core_axis_name="c", subcore_axis_name="s")

    scratch = dict(
        rows=pltpu.VMEM((EDGE_B, D_FEAT), jnp.float32),
        rows2=pltpu.VMEM((EDGE_B, D_FEAT), jnp.float32),
        acc=pltpu.VMEM_SHARED((ACC_ROWS, D_FEAT), jnp.float32),
        gsem=pltpu.SemaphoreType.DMA,
        gsem2=pltpu.SemaphoreType.DMA,
        ssem=pltpu.SemaphoreType.DMA,
        ssem2=pltpu.SemaphoreType.DMA,
    )
    for par in "AB":
        scratch[f"sbuf{par}"] = pltpu.VMEM((CHUNK_B, 1, EDGE_B), jnp.int32)
        scratch[f"dbuf{par}"] = pltpu.VMEM((CHUNK_B, 1, EDGE_B), jnp.int32)
        scratch[f"csem{par}"] = pltpu.SemaphoreType.DMA
        scratch[f"cdsem{par}"] = pltpu.SemaphoreType.DMA

    @functools.partial(
        pl.kernel,
        out_type=jax.ShapeDtypeStruct((NC, N_NODES, D_FEAT), jnp.float32),
        mesh=mesh,
        scratch_types=scratch,
    )
    def kern(x_hbm, s_hbm, d_hbm, part_hbm, *, rows, rows2, acc,
             gsem, gsem2, ssem, ssem2, **cbufs):
        c = lax.axis_index("c")
        s = lax.axis_index("s")
        w = c * NS + s
        ck0 = w * N_CHUNKS_W
        rowbuf = [rows, rows2]
        rsem = [gsem, gsem2]
        wsem = [ssem, ssem2]

        def chunk_fire(k, par):
            pltpu.async_copy(s_hbm.at[ck0 + k], cbufs[f"sbuf{par}"],
                             cbufs[f"csem{par}"])
            pltpu.async_copy(d_hbm.at[ck0 + k], cbufs[f"dbuf{par}"],
                             cbufs[f"cdsem{par}"])

        def chunk_wait(par):
            pltpu.make_async_copy(s_hbm.at[0], cbufs[f"sbuf{par}"],
                                  cbufs[f"csem{par}"]).wait()
            pltpu.make_async_copy(d_hbm.at[0], cbufs[f"dbuf{par}"],
                                  cbufs[f"cdsem{par}"]).wait()

        def scat_drain():
            for b in range(2):
                pltpu.make_async_copy(x_hbm.at[pl.ds(0, EDGE_B), :],
                                      rowbuf[b], wsem[b]).wait()

        chunk_fire(0, "A")

        zero = jnp.zeros((16,), jnp.float32)

        def zrow(r, _):
            def zcol(kk, _):
                rows[r, pl.ds(kk * 16, 16)] = zero
                return 0
            return lax.fori_loop(0, D_FEAT // 16, zcol, 0)

        lax.fori_loop(0, ROW_CHUNK, zrow, 0)

        def zchunk(i, _):
            ch = s + i * NS

            @pl.when(ch < N_CHUNK)
            def _():
                pltpu.async_copy(rows.at[pl.ds(0, ROW_CHUNK), :],
                                 acc.at[pl.ds(ch * ROW_CHUNK, ROW_CHUNK), :],
                                 ssem)
            return 0

        lax.fori_loop(0, CHUNK_PER_S, zchunk, 0)

        @pl.when(s == 0)
        def _():
            pltpu.async_copy(rows.at[pl.ds(0, ROW_CHUNK), :],
                             acc.at[pl.ds(N_NODES, ROW_CHUNK), :], ssem)

        def zdrain(i, _):
            ch = s + i * NS

            @pl.when(ch < N_CHUNK)
            def _():
                pltpu.make_async_copy(rows.at[pl.ds(0, ROW_CHUNK), :],
                                      acc.at[pl.ds(0, ROW_CHUNK), :],
                                      ssem).wait()
            return 0

        lax.fori_loop(0, CHUNK_PER_S, zdrain, 0)

        @pl.when(s == 0)
        def _():
            pltpu.make_async_copy(rows.at[pl.ds(0, ROW_CHUNK), :],
                                  acc.at[pl.ds(0, ROW_CHUNK), :], ssem).wait()
        plsc.subcore_barrier()

        def make_pair(par, first_chunk, next_par):
            sb = cbufs[f"sbuf{par}"]
            db = cbufs[f"dbuf{par}"]

            def pair(p, k):
                base = (ck0 + k) * CHUNK_B + 2 * p
                for b in range(2):
                    if first_chunk:
                        cond = (p > 0) & (base + b - 2 < N_REAL_BATCH)
                    else:
                        cond = base + b - 2 < N_REAL_BATCH

                    @pl.when(cond)
                    def _(b=b):
                        pltpu.make_async_copy(x_hbm.at[pl.ds(0, EDGE_B), :],
                                              rowbuf[b], wsem[b]).wait()

                @pl.when(p == 0)
                def _():
                    if k + 1 < N_CHUNKS_W:
                        chunk_fire(k + 1, next_par)

                gets = []
                for b in range(2):
                    @pl.when(base + b < N_REAL_BATCH)
                    def _(b=b):
                        pltpu.async_copy(x_hbm.at[sb.at[2 * p + b, 0]],
                                         rowbuf[b], rsem[b])
                for b in range(2):
                    @pl.when(base + b < N_REAL_BATCH)
                    def _(b=b):
                        pltpu.make_async_copy(x_hbm.at[pl.ds(0, EDGE_B), :],
                                              rowbuf[b], rsem[b]).wait()
                        pltpu.async_copy(rowbuf[b], acc.at[db.at[2 * p + b, 0]],
                                         wsem[b], add=True)

            return pair

        for k in range(N_CHUNKS_W):
            par = "AB"[k % 2]
            nxt = "AB"[(k + 1) % 2]
            chunk_wait(par)
            pair_fn = make_pair(par, k == 0, nxt)

            def body(p, _, fn=pair_fn, kk=k):
                fn(p, kk)
                return 0

            lax.fori_loop(0, PAIRS_PER_CHUNK, body, 0)

        for b in range(2):
            last = (ck0 + N_CHUNKS_W) * CHUNK_B - 2 + b

            @pl.when(last < N_REAL_BATCH)
            def _(b=b):
                pltpu.make_async_copy(x_hbm.at[pl.ds(0, EDGE_B), :],
                                      rowbuf[b], wsem[b]).wait()
        plsc.subcore_barrier()

        def wchunk(i, _):
            ch = s + i * NS

            @pl.when(ch < N_CHUNK)
            def _():
                r0 = ch * ROW_CHUNK
                pltpu.async_copy(
                    acc.at[pl.ds(r0, ROW_CHUNK), :],
                    part_hbm.at[c, pl.ds(r0, ROW_CHUNK), :],
                    ssem,
                )
            return 0

        lax.fori_loop(0, CHUNK_PER_S, wchunk, 0)

        def wdrain(i, _):
            ch = s + i * NS

            @pl.when(ch < N_CHUNK)
            def _():
                pltpu.make_async_copy(
                    acc.at[pl.ds(0, ROW_CHUNK), :],
                    part_hbm.at[c, pl.ds(0, ROW_CHUNK), :],
                    ssem,
                ).wait()
            return 0

        lax.fori_loop(0, CHUNK_PER_S, wdrain, 0)

    return kern(x, src4d, dst4d)


def _combine(parts):
    blk = 400

    def body(p_ref, o_ref):
        o_ref[...] = p_ref[0] + p_ref[1]

    return pl.pallas_call(
        body,
        grid=(N_NODES // blk,),
        in_specs=[pl.BlockSpec((NC, blk, D_FEAT), lambda i: (0, i, 0))],
        out_specs=pl.BlockSpec((blk, D_FEAT), lambda i: (i, 0)),
        out_shape=jax.ShapeDtypeStruct((N_NODES, D_FEAT), jnp.float32),
    )(parts)


def kernel(x, edge_index):
    ei = edge_index.astype(jnp.int32)
    n_pad = E_PAD - N_EDGES
    src = jnp.concatenate([ei[0], jnp.zeros((n_pad,), jnp.int32)])
    pad_dst = N_NODES + (jnp.arange(n_pad, dtype=jnp.int32) % EDGE_B)
    dst = jnp.concatenate([ei[1], pad_dst])
    n_ck = N_BATCH // CHUNK_B
    src4d = src.reshape(n_ck, CHUNK_B, 1, EDGE_B)
    dst4d = dst.reshape(n_ck, CHUNK_B, 1, EDGE_B)
    parts = _sc_partial(x, src4d, dst4d)
    return _combine(parts)


# speedup vs baseline: 3.4480x; 1.1389x over previous
import functools

import jax
import jax.numpy as jnp
from jax import lax
from jax.experimental import pallas as pl
from jax.experimental.pallas import tpu as pltpu
from jax.experimental.pallas import tpu_sc as plsc

N_NODES = 10000
D_FEAT = 128
N_EDGES = 320000

NC = 2
NS = 16
NW = NC * NS

EDGE_B = 128
N_BATCH = N_EDGES // EDGE_B
BATCH_PER_W = -(-N_BATCH // NW)

ROW_CHUNK = 80
N_CHUNK = N_NODES // ROW_CHUNK
CHUNK_PER_S = -(-N_CHUNK // NS)


def _sc_partial(x, edge_index):
    mesh = plsc.VectorSubcoreMesh(core_axis_name="c", subcore_axis_name="s")

    scratch = dict(
        rows=pltpu.VMEM((EDGE_B, D_FEAT), jnp.float32),
        rows2=pltpu.VMEM((EDGE_B, D_FEAT), jnp.float32),
        acc=pltpu.VMEM_SHARED((N_NODES, D_FEAT), jnp.float32),
        gsem=pltpu.SemaphoreType.DMA,
        gsem2=pltpu.SemaphoreType.DMA,
        ssem=pltpu.SemaphoreType.DMA,
        ssem2=pltpu.SemaphoreType.DMA,
    )
    for par in "AB":
        for b in range(2):
            scratch[f"sidx{b}{par}"] = pltpu.VMEM((EDGE_B,), jnp.int32)
            scratch[f"didx{b}{par}"] = pltpu.VMEM((EDGE_B,), jnp.int32)
            scratch[f"issem{b}{par}"] = pltpu.SemaphoreType.DMA
            scratch[f"idsem{b}{par}"] = pltpu.SemaphoreType.DMA

    @functools.partial(
        pl.kernel,
        out_type=jax.ShapeDtypeStruct((NC, N_NODES, D_FEAT), jnp.float32),
        mesh=mesh,
        scratch_types=scratch,
    )
    def kern(x_hbm, ei_hbm, part_hbm, *, rows, rows2, acc,
             gsem, gsem2, ssem, ssem2, **ibufs):
        c = lax.axis_index("c")
        s = lax.axis_index("s")
        w = c * NS + s
        lim = jnp.minimum((w + 1) * BATCH_PER_W, N_BATCH)
        rowbuf = [rows, rows2]
        rsem = [gsem, gsem2]
        wsem = [ssem, ssem2]

        def idx_fire(t, par):
            for b in range(2):
                bid = w * BATCH_PER_W + 2 * t + b

                @pl.when(bid < lim)
                def _():
                    base = bid * EDGE_B
                    pltpu.async_copy(ei_hbm.at[0, pl.ds(base, EDGE_B)],
                                     ibufs[f"sidx{b}{par}"],
                                     ibufs[f"issem{b}{par}"])
                    pltpu.async_copy(ei_hbm.at[1, pl.ds(base, EDGE_B)],
                                     ibufs[f"didx{b}{par}"],
                                     ibufs[f"idsem{b}{par}"])

        def idx_wait(t, par):
            for b in range(2):
                bid = w * BATCH_PER_W + 2 * t + b

                @pl.when(bid < lim)
                def _():
                    pltpu.make_async_copy(ei_hbm.at[0, pl.ds(0, EDGE_B)],
                                          ibufs[f"sidx{b}{par}"],
                                          ibufs[f"issem{b}{par}"]).wait()
                    pltpu.make_async_copy(ei_hbm.at[1, pl.ds(0, EDGE_B)],
                                          ibufs[f"didx{b}{par}"],
                                          ibufs[f"idsem{b}{par}"]).wait()

        def scat_drain(t):
            for b in range(2):
                bid = w * BATCH_PER_W + 2 * t + b

                @pl.when((t >= 0) & (bid < lim))
                def _():
                    pltpu.make_async_copy(x_hbm.at[pl.ds(0, EDGE_B), :],
                                          rowbuf[b], wsem[b]).wait()

        idx_fire(0, "A")

        zero = jnp.zeros((16,), jnp.float32)

        def zrow(r, _):
            def zcol(k, _):
                rows[r, pl.ds(k * 16, 16)] = zero
                return 0
            return lax.fori_loop(0, D_FEAT // 16, zcol, 0)

        lax.fori_loop(0, ROW_CHUNK, zrow, 0)

        def zchunk(i, _):
            ch = s + i * NS

            @pl.when(ch < N_CHUNK)
            def _():
                pltpu.async_copy(rows.at[pl.ds(0, ROW_CHUNK), :],
                                 acc.at[pl.ds(ch * ROW_CHUNK, ROW_CHUNK), :],
                                 ssem)
            return 0

        lax.fori_loop(0, CHUNK_PER_S, zchunk, 0)

        def zdrain(i, _):
            ch = s + i * NS

            @pl.when(ch < N_CHUNK)
            def _():
                pltpu.make_async_copy(rows.at[pl.ds(0, ROW_CHUNK), :],
                                      acc.at[pl.ds(0, ROW_CHUNK), :],
                                      ssem).wait()
            return 0

        lax.fori_loop(0, CHUNK_PER_S, zdrain, 0)
        plsc.subcore_barrier()

        def pair(t, par, nxt):
            scat_drain(t - 1)
            idx_wait(t, par)
            idx_fire(t + 1, nxt)
            gets = []
            for b in range(2):
                bid = w * BATCH_PER_W + 2 * t + b

                @pl.when(bid < lim)
                def _():
                    pltpu.async_copy(x_hbm.at[ibufs[f"sidx{b}{par}"]],
                                     rowbuf[b], rsem[b])
            for b in range(2):
                bid = w * BATCH_PER_W + 2 * t + b

                @pl.when(bid < lim)
                def _():
                    pltpu.make_async_copy(x_hbm.at[pl.ds(0, EDGE_B), :],
                                          rowbuf[b], rsem[b]).wait()
                    pltpu.async_copy(rowbuf[b], acc.at[ibufs[f"didx{b}{par}"]],
                                     wsem[b], add=True)

        n_pair = (BATCH_PER_W + 1) // 2

        def quad(q, _):
            pair(2 * q, "A", "B")
            pair(2 * q + 1, "B", "A")
            return 0

        lax.fori_loop(0, n_pair // 2, quad, 0)
        scat_drain(n_pair - 1)
        plsc.subcore_barrier()

        def wchunk(i, _):
            ch = s + i * NS

            @pl.when(ch < N_CHUNK)
            def _():
                r0 = ch * ROW_CHUNK
                pltpu.async_copy(
                    acc.at[pl.ds(r0, ROW_CHUNK), :],
                    part_hbm.at[c, pl.ds(r0, ROW_CHUNK), :],
                    ssem,
                )
            return 0

        lax.fori_loop(0, CHUNK_PER_S, wchunk, 0)

        def wdrain(i, _):
            ch = s + i * NS

            @pl.when(ch < N_CHUNK)
            def _():
                pltpu.make_async_copy(
                    acc.at[pl.ds(0, ROW_CHUNK), :],
                    part_hbm.at[c, pl.ds(0, ROW_CHUNK), :],
                    ssem,
                ).wait()
            return 0

        lax.fori_loop(0, CHUNK_PER_S, wdrain, 0)

    return kern(x, edge_index)


def _combine(parts):
    blk = 2000

    def body(p_ref, o_ref):
        o_ref[...] = p_ref[0] + p_ref[1]

    return pl.pallas_call(
        body,
        grid=(N_NODES // blk,),
        in_specs=[pl.BlockSpec((NC, blk, D_FEAT), lambda i: (0, i, 0))],
        out_specs=pl.BlockSpec((blk, D_FEAT), lambda i: (i, 0)),
        out_shape=jax.ShapeDtypeStruct((N_NODES, D_FEAT), jnp.float32),
    )(parts)


def kernel(x, edge_index):
    ei = edge_index.astype(jnp.int32)
    parts = _sc_partial(x, ei)
    return _combine(parts)


# speedup vs baseline: 3.4683x; 1.0059x over previous
import functools

import jax
import jax.numpy as jnp
from jax import lax
from jax.experimental import pallas as pl
from jax.experimental.pallas import tpu as pltpu
from jax.experimental.pallas import tpu_sc as plsc

N_NODES = 10000
D_FEAT = 128
N_EDGES = 320000

NC = 2
NS = 16
NW = NC * NS

EDGE_B = 128
N_BATCH = N_EDGES // EDGE_B
BATCH_PER_W = -(-N_BATCH // NW)

ROW_CHUNK = 80
N_CHUNK = N_NODES // ROW_CHUNK
CHUNK_PER_S = -(-N_CHUNK // NS)


def _sc_partial(x, edge_index):
    mesh = plsc.VectorSubcoreMesh(core_axis_name="c", subcore_axis_name="s")

    scratch = dict(
        rows=pltpu.VMEM((EDGE_B, D_FEAT), jnp.float32),
        rows2=pltpu.VMEM((EDGE_B, D_FEAT), jnp.float32),
        acc=pltpu.VMEM_SHARED((N_NODES, D_FEAT), jnp.float32),
        gsem=pltpu.SemaphoreType.DMA,
        gsem2=pltpu.SemaphoreType.DMA,
        ssem=pltpu.SemaphoreType.DMA,
        ssem2=pltpu.SemaphoreType.DMA,
    )
    for par in "AB":
        for b in range(2):
            scratch[f"sidx{b}{par}"] = pltpu.VMEM((EDGE_B,), jnp.int32)
            scratch[f"didx{b}{par}"] = pltpu.VMEM((EDGE_B,), jnp.int32)
            scratch[f"issem{b}{par}"] = pltpu.SemaphoreType.DMA
            scratch[f"idsem{b}{par}"] = pltpu.SemaphoreType.DMA

    @functools.partial(
        pl.kernel,
        out_type=jax.ShapeDtypeStruct((NC, N_NODES, D_FEAT), jnp.float32),
        mesh=mesh,
        scratch_types=scratch,
    )
    def kern(x_hbm, ei_hbm, part_hbm, *, rows, rows2, acc,
             gsem, gsem2, ssem, ssem2, **ibufs):
        c = lax.axis_index("c")
        s = lax.axis_index("s")
        w = c * NS + s
        lim = jnp.minimum((w + 1) * BATCH_PER_W, N_BATCH)
        rowbuf = [rows, rows2]
        rsem = [gsem, gsem2]
        wsem = [ssem, ssem2]

        def idx_fire(t, par):
            for b in range(2):
                bid = w * BATCH_PER_W + 2 * t + b

                @pl.when(bid < lim)
                def _():
                    base = bid * EDGE_B
                    pltpu.async_copy(ei_hbm.at[0, pl.ds(base, EDGE_B)],
                                     ibufs[f"sidx{b}{par}"],
                                     ibufs[f"issem{b}{par}"])
                    pltpu.async_copy(ei_hbm.at[1, pl.ds(base, EDGE_B)],
                                     ibufs[f"didx{b}{par}"],
                                     ibufs[f"idsem{b}{par}"])

        def idx_wait(t, par):
            for b in range(2):
                bid = w * BATCH_PER_W + 2 * t + b

                @pl.when(bid < lim)
                def _():
                    pltpu.make_async_copy(ei_hbm.at[0, pl.ds(0, EDGE_B)],
                                          ibufs[f"sidx{b}{par}"],
                                          ibufs[f"issem{b}{par}"]).wait()
                    pltpu.make_async_copy(ei_hbm.at[1, pl.ds(0, EDGE_B)],
                                          ibufs[f"didx{b}{par}"],
                                          ibufs[f"idsem{b}{par}"]).wait()

        def scat_drain(t):
            for b in range(2):
                bid = w * BATCH_PER_W + 2 * t + b

                @pl.when((t >= 0) & (bid < lim))
                def _():
                    pltpu.make_async_copy(x_hbm.at[pl.ds(0, EDGE_B), :],
                                          rowbuf[b], wsem[b]).wait()

        idx_fire(0, "A")

        zero = jnp.zeros((16,), jnp.float32)

        def zrow(r, _):
            def zcol(k, _):
                rows[r, pl.ds(k * 16, 16)] = zero
                return 0
            return lax.fori_loop(0, D_FEAT // 16, zcol, 0)

        lax.fori_loop(0, ROW_CHUNK, zrow, 0)

        def zchunk(i, _):
            ch = s + i * NS

            @pl.when(ch < N_CHUNK)
            def _():
                pltpu.async_copy(rows.at[pl.ds(0, ROW_CHUNK), :],
                                 acc.at[pl.ds(ch * ROW_CHUNK, ROW_CHUNK), :],
                                 ssem)
            return 0

        lax.fori_loop(0, CHUNK_PER_S, zchunk, 0)

        def zdrain(i, _):
            ch = s + i * NS

            @pl.when(ch < N_CHUNK)
            def _():
                pltpu.make_async_copy(rows.at[pl.ds(0, ROW_CHUNK), :],
                                      acc.at[pl.ds(0, ROW_CHUNK), :],
                                      ssem).wait()
            return 0

        lax.fori_loop(0, CHUNK_PER_S, zdrain, 0)
        plsc.subcore_barrier()

        def pair(t, par, nxt):
            scat_drain(t - 1)
            idx_wait(t, par)
            idx_fire(t + 1, nxt)
            gets = []
            for b in range(2):
                bid = w * BATCH_PER_W + 2 * t + b

                @pl.when(bid < lim)
                def _():
                    pltpu.async_copy(x_hbm.at[ibufs[f"sidx{b}{par}"]],
                                     rowbuf[b], rsem[b])
            for b in range(2):
                bid = w * BATCH_PER_W + 2 * t + b

                @pl.when(bid < lim)
                def _():
                    pltpu.make_async_copy(x_hbm.at[pl.ds(0, EDGE_B), :],
                                          rowbuf[b], rsem[b]).wait()
                    pltpu.async_copy(rowbuf[b], acc.at[ibufs[f"didx{b}{par}"]],
                                     wsem[b], add=True)

        n_pair = (BATCH_PER_W + 1) // 2

        def quad(q, _):
            pair(2 * q, "A", "B")
            pair(2 * q + 1, "B", "A")
            return 0

        lax.fori_loop(0, n_pair // 2, quad, 0)
        scat_drain(n_pair - 1)
        plsc.subcore_barrier()

        def wchunk(i, _):
            ch = s + i * NS

            @pl.when(ch < N_CHUNK)
            def _():
                r0 = ch * ROW_CHUNK
                pltpu.async_copy(
                    acc.at[pl.ds(r0, ROW_CHUNK), :],
                    part_hbm.at[c, pl.ds(r0, ROW_CHUNK), :],
                    ssem,
                )
            return 0

        lax.fori_loop(0, CHUNK_PER_S, wchunk, 0)

        def wdrain(i, _):
            ch = s + i * NS

            @pl.when(ch < N_CHUNK)
            def _():
                pltpu.make_async_copy(
                    acc.at[pl.ds(0, ROW_CHUNK), :],
                    part_hbm.at[c, pl.ds(0, ROW_CHUNK), :],
                    ssem,
                ).wait()
            return 0

        lax.fori_loop(0, CHUNK_PER_S, wdrain, 0)

    return kern(x, edge_index)


def _combine(parts):
    blk = 10000

    def body(p_ref, o_ref):
        o_ref[...] = p_ref[0] + p_ref[1]

    return pl.pallas_call(
        body,
        grid=(N_NODES // blk,),
        in_specs=[pl.BlockSpec((NC, blk, D_FEAT), lambda i: (0, i, 0))],
        out_specs=pl.BlockSpec((blk, D_FEAT), lambda i: (i, 0)),
        out_shape=jax.ShapeDtypeStruct((N_NODES, D_FEAT), jnp.float32),
    )(parts)


def kernel(x, edge_index):
    ei = edge_index.astype(jnp.int32)
    parts = _sc_partial(x, ei)
    return _combine(parts)
